# baseline (speedup 1.0000x reference)
import functools

import jax
import jax.numpy as jnp
from jax import lax
from jax.experimental import pallas as pl
from jax.experimental.pallas import tpu as pltpu
from jax.experimental.pallas import tpu_sc as plsc

MIN_NORM = 1e-15
N = 10000
D = 128
E = 320000
NC = 2
NS = 16
CHUNK = 128
NBLK = 2
KB = 40
EDGES_PER_TILE = E // (NC * NS)
EPT_PAD = NBLK * KB * CHUNK
NPAD = 10240
ROWS_PER_TILE = NPAD // NS
ROW_BLK = 1000


def _artanh(x):
    x = jnp.clip(x, -1.0 + 1e-7, 1.0 - 1e-7)
    return 0.5 * jnp.log((1.0 + x) / (1.0 - x))


def _proj(x):
    norm = jnp.maximum(
        jnp.sqrt(jnp.sum(x * x, axis=-1, keepdims=True)), MIN_NORM)
    maxnorm = 1.0 - 4e-3
    return jnp.where(norm > maxnorm, x / norm * maxnorm, x)


def _dense_body(x_ref, w_ref, b_ref, out_ref):
    x = x_ref[...]
    w = w_ref[...]
    b = b_ref[...]
    x_norm = jnp.maximum(
        jnp.sqrt(jnp.sum(x * x, axis=-1, keepdims=True)), MIN_NORM)
    mx = jnp.dot(x, w, preferred_element_type=jnp.float32)
    mx_norm = jnp.maximum(
        jnp.sqrt(jnp.sum(mx * mx, axis=-1, keepdims=True)), MIN_NORM)
    res = jnp.tanh(mx_norm / x_norm * _artanh(x_norm)) * mx / mx_norm
    support = jnp.where(jnp.all(mx == 0.0, axis=-1, keepdims=True), 0.0, res)
    b_norm = jnp.maximum(
        jnp.sqrt(jnp.sum(b * b, axis=-1, keepdims=True)), MIN_NORM)
    hb = _proj(jnp.tanh(b_norm) * b / b_norm)
    x2 = jnp.sum(support * support, axis=-1, keepdims=True)
    y2 = jnp.sum(hb * hb, axis=-1, keepdims=True)
    xy = jnp.sum(support * hb, axis=-1, keepdims=True)
    num = (1.0 + 2.0 * xy + y2) * support + (1.0 - x2) * hb
    den = 1.0 + 2.0 * xy + x2 * y2
    s = _proj(num / jnp.maximum(den, MIN_NORM))
    p_norm = jnp.maximum(
        jnp.sqrt(jnp.sum(s * s, axis=-1, keepdims=True)), MIN_NORM)
    out_ref[...] = (_artanh(p_norm) / p_norm) * s


_dense = pl.pallas_call(
    _dense_body,
    grid=(N // ROW_BLK,),
    in_specs=[
        pl.BlockSpec((ROW_BLK, D), lambda i: (i, 0)),
        pl.BlockSpec((D, D), lambda i: (0, 0)),
        pl.BlockSpec((1, D), lambda i: (0, 0)),
    ],
    out_specs=pl.BlockSpec((ROW_BLK, D), lambda i: (i, 0)),
    out_shape=jax.ShapeDtypeStruct((N, D), jnp.float32),
)


def _sc_body(st_hbm, src_hbm, dst_hbm, zeros_hbm, out_hbm,
             src_v, dst_v, rows0, rows1, acc, sem):
    cid = lax.axis_index("c")
    sid = lax.axis_index("s")
    row0 = sid * ROWS_PER_TILE
    pltpu.sync_copy(zeros_hbm, acc.at[pl.ds(row0, ROWS_PER_TILE)])
    plsc.subcore_barrier()

    def step(k, carry):
        pltpu.sync_copy(src_hbm.at[cid, sid, k], src_v)
        pltpu.sync_copy(dst_hbm.at[cid, sid, k], dst_v)
        pltpu.async_copy(st_hbm.at[src_v], rows0, sem).wait()
        pltpu.sync_copy(rows0, acc.at[dst_v], add=True)
        return carry

    lax.fori_loop(0, NBLK * KB, step, 0)

    plsc.subcore_barrier()
    pltpu.sync_copy(acc.at[pl.ds(row0, ROWS_PER_TILE)],
                    out_hbm.at[cid, pl.ds(row0, ROWS_PER_TILE)])


@functools.cache
def _sc_spmm():
    return pl.kernel(
        _sc_body,
        out_type=jax.ShapeDtypeStruct((NC, NPAD, D), jnp.float32),
        mesh=plsc.VectorSubcoreMesh(core_axis_name="c", subcore_axis_name="s",
                                    num_cores=NC, num_subcores=NS),
        scratch_types=[
            pltpu.VMEM((CHUNK,), jnp.int32),
            pltpu.VMEM((CHUNK,), jnp.int32),
            pltpu.VMEM((CHUNK, D), jnp.float32),
            pltpu.VMEM((CHUNK, D), jnp.float32),
            pltpu.VMEM_SHARED((NPAD, D), jnp.float32),
            pltpu.SemaphoreType.DMA,
        ],
    )


def _add_relu_body(p_ref, out_ref):
    out_ref[...] = jnp.maximum(p_ref[0] + p_ref[1], 0.0)


_add_relu = pl.pallas_call(
    _add_relu_body,
    grid=(N // ROW_BLK,),
    in_specs=[pl.BlockSpec((NC, ROW_BLK, D), lambda i: (0, i, 0))],
    out_specs=pl.BlockSpec((ROW_BLK, D), lambda i: (i, 0)),
    out_shape=jax.ShapeDtypeStruct((N, D), jnp.float32),
)


def kernel(adjacency_edge_index, input_feature, weight, bias):
    st = _dense(input_feature, weight, bias.reshape(1, D))
    ei = adjacency_edge_index.astype(jnp.int32)
    npt = EPT_PAD - EDGES_PER_TILE
    dst_i = jnp.pad(ei[0].reshape(NC * NS, EDGES_PER_TILE),
                    ((0, 0), (0, npt)), constant_values=NPAD - 1)
    src_i = jnp.pad(ei[1].reshape(NC * NS, EDGES_PER_TILE),
                    ((0, 0), (0, npt)), constant_values=0)
    dst_i = dst_i.reshape(NC, NS, NBLK * KB, CHUNK)
    src_i = src_i.reshape(NC, NS, NBLK * KB, CHUNK)
    zeros = jnp.zeros((ROWS_PER_TILE, D), jnp.float32)
    partials = _sc_spmm()(st, src_i, dst_i, zeros)
    return _add_relu(partials)


# speedup vs baseline: 1.7132x; 1.7132x over previous
import functools

import jax
import jax.numpy as jnp
from jax import lax
from jax.experimental import pallas as pl
from jax.experimental.pallas import tpu as pltpu
from jax.experimental.pallas import tpu_sc as plsc

MIN_NORM = 1e-15
N = 10000
D = 128
E = 320000
NC = 2
NS = 16
CHUNK = 100
EDGES_PER_TILE = E // (NC * NS)
K = EDGES_PER_TILE // CHUNK
NPAD = 10240
ROWS_PER_TILE = NPAD // NS
ROW_BLK = 1000


def _artanh(x):
    x = jnp.clip(x, -1.0 + 1e-7, 1.0 - 1e-7)
    return 0.5 * jnp.log((1.0 + x) / (1.0 - x))


def _proj(x):
    norm = jnp.maximum(
        jnp.sqrt(jnp.sum(x * x, axis=-1, keepdims=True)), MIN_NORM)
    maxnorm = 1.0 - 4e-3
    return jnp.where(norm > maxnorm, x / norm * maxnorm, x)


def _dense_body(x_ref, w_ref, b_ref, out_ref):
    x = x_ref[...]
    w = w_ref[...]
    b = b_ref[...]
    x_norm = jnp.maximum(
        jnp.sqrt(jnp.sum(x * x, axis=-1, keepdims=True)), MIN_NORM)
    mx = jnp.dot(x, w, preferred_element_type=jnp.float32)
    mx_norm = jnp.maximum(
        jnp.sqrt(jnp.sum(mx * mx, axis=-1, keepdims=True)), MIN_NORM)
    res = jnp.tanh(mx_norm / x_norm * _artanh(x_norm)) * mx / mx_norm
    support = jnp.where(jnp.all(mx == 0.0, axis=-1, keepdims=True), 0.0, res)
    b_norm = jnp.maximum(
        jnp.sqrt(jnp.sum(b * b, axis=-1, keepdims=True)), MIN_NORM)
    hb = _proj(jnp.tanh(b_norm) * b / b_norm)
    x2 = jnp.sum(support * support, axis=-1, keepdims=True)
    y2 = jnp.sum(hb * hb, axis=-1, keepdims=True)
    xy = jnp.sum(support * hb, axis=-1, keepdims=True)
    num = (1.0 + 2.0 * xy + y2) * support + (1.0 - x2) * hb
    den = 1.0 + 2.0 * xy + x2 * y2
    s = _proj(num / jnp.maximum(den, MIN_NORM))
    p_norm = jnp.maximum(
        jnp.sqrt(jnp.sum(s * s, axis=-1, keepdims=True)), MIN_NORM)
    out_ref[...] = (_artanh(p_norm) / p_norm) * s


_dense = pl.pallas_call(
    _dense_body,
    grid=(N // ROW_BLK,),
    in_specs=[
        pl.BlockSpec((ROW_BLK, D), lambda i: (i, 0)),
        pl.BlockSpec((D, D), lambda i: (0, 0)),
        pl.BlockSpec((1, D), lambda i: (0, 0)),
    ],
    out_specs=pl.BlockSpec((ROW_BLK, D), lambda i: (i, 0)),
    out_shape=jax.ShapeDtypeStruct((N, D), jnp.float32),
)


def _sc_body(st_hbm, src_hbm, dst_hbm, zeros_hbm, out_hbm,
             src_v, dst_v, rows0, rows1, acc, sem):
    cid = lax.axis_index("c")
    sid = lax.axis_index("s")
    row0 = sid * ROWS_PER_TILE
    pltpu.sync_copy(zeros_hbm, acc.at[pl.ds(row0, ROWS_PER_TILE)])
    plsc.subcore_barrier()

    def step(k, carry):
        pltpu.sync_copy(src_hbm.at[cid, sid, k], src_v)
        pltpu.sync_copy(dst_hbm.at[cid, sid, k], dst_v)
        pltpu.async_copy(st_hbm.at[src_v], rows0, sem).wait()
        pltpu.sync_copy(rows0, acc.at[dst_v], add=True)
        return carry

    lax.fori_loop(0, K, step, 0)

    plsc.subcore_barrier()
    pltpu.sync_copy(acc.at[pl.ds(row0, ROWS_PER_TILE)],
                    out_hbm.at[cid, pl.ds(row0, ROWS_PER_TILE)])


@functools.cache
def _sc_spmm():
    return pl.kernel(
        _sc_body,
        out_type=jax.ShapeDtypeStruct((NC, NPAD, D), jnp.float32),
        mesh=plsc.VectorSubcoreMesh(core_axis_name="c", subcore_axis_name="s",
                                    num_cores=NC, num_subcores=NS),
        scratch_types=[
            pltpu.VMEM((CHUNK,), jnp.int32),
            pltpu.VMEM((CHUNK,), jnp.int32),
            pltpu.VMEM((CHUNK, D), jnp.float32),
            pltpu.VMEM((CHUNK, D), jnp.float32),
            pltpu.VMEM_SHARED((NPAD, D), jnp.float32),
            pltpu.SemaphoreType.DMA,
        ],
    )


def _add_relu_body(p_ref, out_ref):
    out_ref[...] = jnp.maximum(p_ref[0] + p_ref[1], 0.0)


_add_relu = pl.pallas_call(
    _add_relu_body,
    grid=(N // ROW_BLK,),
    in_specs=[pl.BlockSpec((NC, ROW_BLK, D), lambda i: (0, i, 0))],
    out_specs=pl.BlockSpec((ROW_BLK, D), lambda i: (i, 0)),
    out_shape=jax.ShapeDtypeStruct((N, D), jnp.float32),
)


def kernel(adjacency_edge_index, input_feature, weight, bias):
    st = _dense(input_feature, weight, bias.reshape(1, D))
    ei = adjacency_edge_index.astype(jnp.int32)
    dst_i = ei[0].reshape(NC, NS, K, CHUNK)
    src_i = ei[1].reshape(NC, NS, K, CHUNK)
    zeros = jnp.zeros((ROWS_PER_TILE, D), jnp.float32)
    partials = _sc_spmm()(st, src_i, dst_i, zeros)
    return _add_relu(partials)


# speedup vs baseline: 2.2939x; 1.3390x over previous
import functools

import jax
import jax.numpy as jnp
from jax import lax
from jax.experimental import pallas as pl
from jax.experimental.pallas import tpu as pltpu
from jax.experimental.pallas import tpu_sc as plsc

MIN_NORM = 1e-15
N = 10000
D = 128
E = 320000
NC = 2
NS = 16
CHUNK = 100
EDGES_PER_TILE = E // (NC * NS)
K = EDGES_PER_TILE // CHUNK
NBLK = 2
KB = K // NBLK
NPAD = 10240
ROWS_PER_TILE = NPAD // NS
ROW_BLK = 1000


def _artanh(x):
    x = jnp.clip(x, -1.0 + 1e-7, 1.0 - 1e-7)
    return 0.5 * jnp.log((1.0 + x) / (1.0 - x))


def _proj(x):
    norm = jnp.maximum(
        jnp.sqrt(jnp.sum(x * x, axis=-1, keepdims=True)), MIN_NORM)
    maxnorm = 1.0 - 4e-3
    return jnp.where(norm > maxnorm, x / norm * maxnorm, x)


def _dense_body(x_ref, w_ref, b_ref, out_ref):
    x = x_ref[...]
    w = w_ref[...]
    b = b_ref[...]
    x_norm = jnp.maximum(
        jnp.sqrt(jnp.sum(x * x, axis=-1, keepdims=True)), MIN_NORM)
    mx = jnp.dot(x, w, preferred_element_type=jnp.float32)
    mx_norm = jnp.maximum(
        jnp.sqrt(jnp.sum(mx * mx, axis=-1, keepdims=True)), MIN_NORM)
    res = jnp.tanh(mx_norm / x_norm * _artanh(x_norm)) * mx / mx_norm
    support = jnp.where(jnp.all(mx == 0.0, axis=-1, keepdims=True), 0.0, res)
    b_norm = jnp.maximum(
        jnp.sqrt(jnp.sum(b * b, axis=-1, keepdims=True)), MIN_NORM)
    hb = _proj(jnp.tanh(b_norm) * b / b_norm)
    x2 = jnp.sum(support * support, axis=-1, keepdims=True)
    y2 = jnp.sum(hb * hb, axis=-1, keepdims=True)
    xy = jnp.sum(support * hb, axis=-1, keepdims=True)
    num = (1.0 + 2.0 * xy + y2) * support + (1.0 - x2) * hb
    den = 1.0 + 2.0 * xy + x2 * y2
    s = _proj(num / jnp.maximum(den, MIN_NORM))
    p_norm = jnp.maximum(
        jnp.sqrt(jnp.sum(s * s, axis=-1, keepdims=True)), MIN_NORM)
    out_ref[...] = (_artanh(p_norm) / p_norm) * s


_dense = pl.pallas_call(
    _dense_body,
    grid=(N // ROW_BLK,),
    in_specs=[
        pl.BlockSpec((ROW_BLK, D), lambda i: (i, 0)),
        pl.BlockSpec((D, D), lambda i: (0, 0)),
        pl.BlockSpec((1, D), lambda i: (0, 0)),
    ],
    out_specs=pl.BlockSpec((ROW_BLK, D), lambda i: (i, 0)),
    out_shape=jax.ShapeDtypeStruct((N, D), jnp.float32),
)


def _sc_body(st_hbm, src_hbm, dst_hbm, zeros_hbm, out_hbm,
             src_v, dst_v, rows0, rows1, acc, sem):
    cid = lax.axis_index("c")
    sid = lax.axis_index("s")
    row0 = sid * ROWS_PER_TILE
    pltpu.sync_copy(zeros_hbm, acc.at[pl.ds(row0, ROWS_PER_TILE)])
    plsc.subcore_barrier()

    for blk in range(NBLK):
        pltpu.sync_copy(src_hbm.at[cid, sid, blk], src_v)
        pltpu.sync_copy(dst_hbm.at[cid, sid, blk], dst_v)

        def step(k, carry):
            pltpu.async_copy(st_hbm.at[src_v.at[k]], rows0, sem).wait()
            pltpu.sync_copy(rows0, acc.at[dst_v.at[k]], add=True)
            return carry

        lax.fori_loop(0, KB, step, 0)

    plsc.subcore_barrier()
    pltpu.sync_copy(acc.at[pl.ds(row0, ROWS_PER_TILE)],
                    out_hbm.at[cid, pl.ds(row0, ROWS_PER_TILE)])


@functools.cache
def _sc_spmm():
    return pl.kernel(
        _sc_body,
        out_type=jax.ShapeDtypeStruct((NC, NPAD, D), jnp.float32),
        mesh=plsc.VectorSubcoreMesh(core_axis_name="c", subcore_axis_name="s",
                                    num_cores=NC, num_subcores=NS),
        scratch_types=[
            pltpu.VMEM((KB, CHUNK), jnp.int32),
            pltpu.VMEM((KB, CHUNK), jnp.int32),
            pltpu.VMEM((CHUNK, D), jnp.float32),
            pltpu.VMEM((CHUNK, D), jnp.float32),
            pltpu.VMEM_SHARED((NPAD, D), jnp.float32),
            pltpu.SemaphoreType.DMA,
        ],
    )


def _add_relu_body(p_ref, out_ref):
    out_ref[...] = jnp.maximum(p_ref[0] + p_ref[1], 0.0)


_add_relu = pl.pallas_call(
    _add_relu_body,
    grid=(N // ROW_BLK,),
    in_specs=[pl.BlockSpec((NC, ROW_BLK, D), lambda i: (0, i, 0))],
    out_specs=pl.BlockSpec((ROW_BLK, D), lambda i: (i, 0)),
    out_shape=jax.ShapeDtypeStruct((N, D), jnp.float32),
)


def kernel(adjacency_edge_index, input_feature, weight, bias):
    st = _dense(input_feature, weight, bias.reshape(1, D))
    ei = adjacency_edge_index.astype(jnp.int32)
    dst_i = ei[0].reshape(NC, NS, NBLK, KB, CHUNK)
    src_i = ei[1].reshape(NC, NS, NBLK, KB, CHUNK)
    zeros = jnp.zeros((ROWS_PER_TILE, D), jnp.float32)
    partials = _sc_spmm()(st, src_i, dst_i, zeros)
    return _add_relu(partials)


# speedup vs baseline: 2.8221x; 1.2302x over previous
import functools

import jax
import jax.numpy as jnp
from jax import lax
from jax.experimental import pallas as pl
from jax.experimental.pallas import tpu as pltpu
from jax.experimental.pallas import tpu_sc as plsc

MIN_NORM = 1e-15
N = 10000
D = 128
E = 320000
NC = 2
NS = 16
CHUNK = 100
EDGES_PER_TILE = E // (NC * NS)
K = EDGES_PER_TILE // CHUNK
NBLK = 2
KB = K // NBLK
NPAD = 10240
ROWS_PER_TILE = NPAD // NS
ROW_BLK = 1000


def _artanh(x):
    x = jnp.clip(x, -1.0 + 1e-7, 1.0 - 1e-7)
    return 0.5 * jnp.log((1.0 + x) / (1.0 - x))


def _proj(x):
    norm = jnp.maximum(
        jnp.sqrt(jnp.sum(x * x, axis=-1, keepdims=True)), MIN_NORM)
    maxnorm = 1.0 - 4e-3
    return jnp.where(norm > maxnorm, x / norm * maxnorm, x)


def _dense_body(x_ref, w_ref, b_ref, out_ref):
    x = x_ref[...]
    w = w_ref[...]
    b = b_ref[...]
    x_norm = jnp.maximum(
        jnp.sqrt(jnp.sum(x * x, axis=-1, keepdims=True)), MIN_NORM)
    mx = jnp.dot(x, w, preferred_element_type=jnp.float32)
    mx_norm = jnp.maximum(
        jnp.sqrt(jnp.sum(mx * mx, axis=-1, keepdims=True)), MIN_NORM)
    res = jnp.tanh(mx_norm / x_norm * _artanh(x_norm)) * mx / mx_norm
    support = jnp.where(jnp.all(mx == 0.0, axis=-1, keepdims=True), 0.0, res)
    b_norm = jnp.maximum(
        jnp.sqrt(jnp.sum(b * b, axis=-1, keepdims=True)), MIN_NORM)
    hb = _proj(jnp.tanh(b_norm) * b / b_norm)
    x2 = jnp.sum(support * support, axis=-1, keepdims=True)
    y2 = jnp.sum(hb * hb, axis=-1, keepdims=True)
    xy = jnp.sum(support * hb, axis=-1, keepdims=True)
    num = (1.0 + 2.0 * xy + y2) * support + (1.0 - x2) * hb
    den = 1.0 + 2.0 * xy + x2 * y2
    s = _proj(num / jnp.maximum(den, MIN_NORM))
    p_norm = jnp.maximum(
        jnp.sqrt(jnp.sum(s * s, axis=-1, keepdims=True)), MIN_NORM)
    out_ref[...] = (_artanh(p_norm) / p_norm) * s


_dense = pl.pallas_call(
    _dense_body,
    grid=(N // ROW_BLK,),
    in_specs=[
        pl.BlockSpec((ROW_BLK, D), lambda i: (i, 0)),
        pl.BlockSpec((D, D), lambda i: (0, 0)),
        pl.BlockSpec((1, D), lambda i: (0, 0)),
    ],
    out_specs=pl.BlockSpec((ROW_BLK, D), lambda i: (i, 0)),
    out_shape=jax.ShapeDtypeStruct((N, D), jnp.float32),
)


def _sc_body(st_hbm, src_hbm, dst_hbm, zeros_hbm, out_hbm,
             src_v, dst_v, rows0, rows1, acc, gsem, ssem):
    cid = lax.axis_index("c")
    sid = lax.axis_index("s")
    row0 = sid * ROWS_PER_TILE
    pltpu.sync_copy(zeros_hbm, acc.at[pl.ds(row0, ROWS_PER_TILE)])
    plsc.subcore_barrier()

    rows = (rows0, rows1)

    def gather(k, b):
        pltpu.async_copy(st_hbm.at[src_v.at[k]], rows[b], gsem).wait()

    def scatter_start(k, b):
        pltpu.async_copy(rows[b], acc.at[dst_v.at[k]], ssem, add=True)

    def scatter_wait(k, b):
        pltpu.make_async_copy(rows[b], acc.at[dst_v.at[k]], ssem).wait()

    for blk in range(NBLK):
        pltpu.sync_copy(src_hbm.at[cid, sid, blk], src_v)
        pltpu.sync_copy(dst_hbm.at[cid, sid, blk], dst_v)

        gather(0, 0)
        scatter_start(0, 0)
        gather(1, 1)
        scatter_start(1, 1)

        def pair(i, carry):
            for b in range(2):
                k = 2 * i + b
                scatter_wait(k - 2, b)
                gather(k, b)
                scatter_start(k, b)
            return carry

        lax.fori_loop(1, KB // 2, pair, 0)
        scatter_wait(KB - 2, 0)
        scatter_wait(KB - 1, 1)

    plsc.subcore_barrier()
    pltpu.sync_copy(acc.at[pl.ds(row0, ROWS_PER_TILE)],
                    out_hbm.at[cid, pl.ds(row0, ROWS_PER_TILE)])


@functools.cache
def _sc_spmm():
    return pl.kernel(
        _sc_body,
        out_type=jax.ShapeDtypeStruct((NC, NPAD, D), jnp.float32),
        mesh=plsc.VectorSubcoreMesh(core_axis_name="c", subcore_axis_name="s",
                                    num_cores=NC, num_subcores=NS),
        scratch_types=[
            pltpu.VMEM((KB, CHUNK), jnp.int32),
            pltpu.VMEM((KB, CHUNK), jnp.int32),
            pltpu.VMEM((CHUNK, D), jnp.float32),
            pltpu.VMEM((CHUNK, D), jnp.float32),
            pltpu.VMEM_SHARED((NPAD, D), jnp.float32),
            pltpu.SemaphoreType.DMA,
            pltpu.SemaphoreType.DMA,
        ],
    )


def _add_relu_body(p_ref, out_ref):
    out_ref[...] = jnp.maximum(p_ref[0] + p_ref[1], 0.0)


_add_relu = pl.pallas_call(
    _add_relu_body,
    grid=(N // ROW_BLK,),
    in_specs=[pl.BlockSpec((NC, ROW_BLK, D), lambda i: (0, i, 0))],
    out_specs=pl.BlockSpec((ROW_BLK, D), lambda i: (i, 0)),
    out_shape=jax.ShapeDtypeStruct((N, D), jnp.float32),
)


def kernel(adjacency_edge_index, input_feature, weight, bias):
    st = _dense(input_feature, weight, bias.reshape(1, D))
    ei = adjacency_edge_index.astype(jnp.int32)
    dst_i = ei[0].reshape(NC, NS, NBLK, KB, CHUNK)
    src_i = ei[1].reshape(NC, NS, NBLK, KB, CHUNK)
    zeros = jnp.zeros((ROWS_PER_TILE, D), jnp.float32)
    partials = _sc_spmm()(st, src_i, dst_i, zeros)
    return _add_relu(partials)


# speedup vs baseline: 2.9485x; 1.0448x over previous
import functools

import jax
import jax.numpy as jnp
from jax import lax
from jax.experimental import pallas as pl
from jax.experimental.pallas import tpu as pltpu
from jax.experimental.pallas import tpu_sc as plsc

MIN_NORM = 1e-15
N = 10000
D = 128
E = 320000
NC = 2
NS = 16
CHUNK = 125
EDGES_PER_TILE = E // (NC * NS)
K = EDGES_PER_TILE // CHUNK
NBLK = 2
KB = K // NBLK
NPAD = 10240
ROWS_PER_TILE = NPAD // NS
ROW_BLK = 1000


def _artanh(x):
    x = jnp.clip(x, -1.0 + 1e-7, 1.0 - 1e-7)
    return 0.5 * jnp.log((1.0 + x) / (1.0 - x))


def _proj(x):
    norm = jnp.maximum(
        jnp.sqrt(jnp.sum(x * x, axis=-1, keepdims=True)), MIN_NORM)
    maxnorm = 1.0 - 4e-3
    return jnp.where(norm > maxnorm, x / norm * maxnorm, x)


def _dense_body(x_ref, w_ref, b_ref, out_ref):
    x = x_ref[...]
    w = w_ref[...]
    b = b_ref[...]
    x_norm = jnp.maximum(
        jnp.sqrt(jnp.sum(x * x, axis=-1, keepdims=True)), MIN_NORM)
    mx = jnp.dot(x, w, preferred_element_type=jnp.float32)
    mx_norm = jnp.maximum(
        jnp.sqrt(jnp.sum(mx * mx, axis=-1, keepdims=True)), MIN_NORM)
    res = jnp.tanh(mx_norm / x_norm * _artanh(x_norm)) * mx / mx_norm
    support = jnp.where(jnp.all(mx == 0.0, axis=-1, keepdims=True), 0.0, res)
    b_norm = jnp.maximum(
        jnp.sqrt(jnp.sum(b * b, axis=-1, keepdims=True)), MIN_NORM)
    hb = _proj(jnp.tanh(b_norm) * b / b_norm)
    x2 = jnp.sum(support * support, axis=-1, keepdims=True)
    y2 = jnp.sum(hb * hb, axis=-1, keepdims=True)
    xy = jnp.sum(support * hb, axis=-1, keepdims=True)
    num = (1.0 + 2.0 * xy + y2) * support + (1.0 - x2) * hb
    den = 1.0 + 2.0 * xy + x2 * y2
    s = _proj(num / jnp.maximum(den, MIN_NORM))
    p_norm = jnp.maximum(
        jnp.sqrt(jnp.sum(s * s, axis=-1, keepdims=True)), MIN_NORM)
    out_ref[...] = (_artanh(p_norm) / p_norm) * s


_dense = pl.pallas_call(
    _dense_body,
    grid=(N // ROW_BLK,),
    in_specs=[
        pl.BlockSpec((ROW_BLK, D), lambda i: (i, 0)),
        pl.BlockSpec((D, D), lambda i: (0, 0)),
        pl.BlockSpec((1, D), lambda i: (0, 0)),
    ],
    out_specs=pl.BlockSpec((ROW_BLK, D), lambda i: (i, 0)),
    out_shape=jax.ShapeDtypeStruct((N, D), jnp.float32),
)


def _sc_body(st_hbm, src_hbm, dst_hbm, zeros_hbm, out_hbm,
             src_v, dst_v, rows0, rows1, acc, gsem, ssem):
    cid = lax.axis_index("c")
    sid = lax.axis_index("s")
    row0 = sid * ROWS_PER_TILE
    pltpu.sync_copy(zeros_hbm, acc.at[pl.ds(row0, ROWS_PER_TILE)])
    plsc.subcore_barrier()

    rows = (rows0, rows1)

    def gather(k, b):
        pltpu.async_copy(st_hbm.at[src_v.at[k]], rows[b], gsem).wait()

    def scatter_start(k, b):
        pltpu.async_copy(rows[b], acc.at[dst_v.at[k]], ssem, add=True)

    def scatter_wait(k, b):
        pltpu.make_async_copy(rows[b], acc.at[dst_v.at[k]], ssem).wait()

    for blk in range(NBLK):
        pltpu.sync_copy(src_hbm.at[cid, sid, blk], src_v)
        pltpu.sync_copy(dst_hbm.at[cid, sid, blk], dst_v)

        gather(0, 0)
        scatter_start(0, 0)
        gather(1, 1)
        scatter_start(1, 1)

        def pair(i, carry):
            for b in range(2):
                k = 2 * i + b
                scatter_wait(k - 2, b)
                gather(k, b)
                scatter_start(k, b)
            return carry

        lax.fori_loop(1, KB // 2, pair, 0)
        scatter_wait(KB - 2, 0)
        scatter_wait(KB - 1, 1)

    plsc.subcore_barrier()
    pltpu.sync_copy(acc.at[pl.ds(row0, ROWS_PER_TILE)],
                    out_hbm.at[cid, pl.ds(row0, ROWS_PER_TILE)])


@functools.cache
def _sc_spmm():
    return pl.kernel(
        _sc_body,
        out_type=jax.ShapeDtypeStruct((NC, NPAD, D), jnp.float32),
        mesh=plsc.VectorSubcoreMesh(core_axis_name="c", subcore_axis_name="s",
                                    num_cores=NC, num_subcores=NS),
        scratch_types=[
            pltpu.VMEM((KB, CHUNK), jnp.int32),
            pltpu.VMEM((KB, CHUNK), jnp.int32),
            pltpu.VMEM((CHUNK, D), jnp.float32),
            pltpu.VMEM((CHUNK, D), jnp.float32),
            pltpu.VMEM_SHARED((NPAD, D), jnp.float32),
            pltpu.SemaphoreType.DMA,
            pltpu.SemaphoreType.DMA,
        ],
    )


def _add_relu_body(p_ref, out_ref):
    out_ref[...] = jnp.maximum(p_ref[0] + p_ref[1], 0.0)


_add_relu = pl.pallas_call(
    _add_relu_body,
    grid=(N // ROW_BLK,),
    in_specs=[pl.BlockSpec((NC, ROW_BLK, D), lambda i: (0, i, 0))],
    out_specs=pl.BlockSpec((ROW_BLK, D), lambda i: (i, 0)),
    out_shape=jax.ShapeDtypeStruct((N, D), jnp.float32),
)


def kernel(adjacency_edge_index, input_feature, weight, bias):
    st = _dense(input_feature, weight, bias.reshape(1, D))
    ei = adjacency_edge_index.astype(jnp.int32)
    dst_i = ei[0].reshape(NC, NS, NBLK, KB, CHUNK)
    src_i = ei[1].reshape(NC, NS, NBLK, KB, CHUNK)
    zeros = jnp.zeros((ROWS_PER_TILE, D), jnp.float32)
    partials = _sc_spmm()(st, src_i, dst_i, zeros)
    return _add_relu(partials)


# speedup vs baseline: 2.9606x; 1.0041x over previous
import functools

import jax
import jax.numpy as jnp
from jax import lax
from jax.experimental import pallas as pl
from jax.experimental.pallas import tpu as pltpu
from jax.experimental.pallas import tpu_sc as plsc

MIN_NORM = 1e-15
N = 10000
D = 128
E = 320000
NC = 2
NS = 16
CHUNK = 125
EDGES_PER_TILE = E // (NC * NS)
K = EDGES_PER_TILE // CHUNK
NBLK = 2
KB = K // NBLK
NPAD = 10240
ROWS_PER_TILE = NPAD // NS
ROW_BLK = 1000


def _artanh(x):
    x = jnp.clip(x, -1.0 + 1e-7, 1.0 - 1e-7)
    return 0.5 * jnp.log((1.0 + x) / (1.0 - x))


def _proj(x):
    norm = jnp.maximum(
        jnp.sqrt(jnp.sum(x * x, axis=-1, keepdims=True)), MIN_NORM)
    maxnorm = 1.0 - 4e-3
    return jnp.where(norm > maxnorm, x / norm * maxnorm, x)


def _dense_body(x_ref, w_ref, b_ref, out_ref):
    x = x_ref[...]
    w = w_ref[...]
    b = b_ref[...]
    x_norm = jnp.maximum(
        jnp.sqrt(jnp.sum(x * x, axis=-1, keepdims=True)), MIN_NORM)
    mx = jnp.dot(x, w, preferred_element_type=jnp.float32)
    mx_norm = jnp.maximum(
        jnp.sqrt(jnp.sum(mx * mx, axis=-1, keepdims=True)), MIN_NORM)
    res = jnp.tanh(mx_norm / x_norm * _artanh(x_norm)) * mx / mx_norm
    support = jnp.where(jnp.all(mx == 0.0, axis=-1, keepdims=True), 0.0, res)
    b_norm = jnp.maximum(
        jnp.sqrt(jnp.sum(b * b, axis=-1, keepdims=True)), MIN_NORM)
    hb = _proj(jnp.tanh(b_norm) * b / b_norm)
    x2 = jnp.sum(support * support, axis=-1, keepdims=True)
    y2 = jnp.sum(hb * hb, axis=-1, keepdims=True)
    xy = jnp.sum(support * hb, axis=-1, keepdims=True)
    num = (1.0 + 2.0 * xy + y2) * support + (1.0 - x2) * hb
    den = 1.0 + 2.0 * xy + x2 * y2
    s = _proj(num / jnp.maximum(den, MIN_NORM))
    p_norm = jnp.maximum(
        jnp.sqrt(jnp.sum(s * s, axis=-1, keepdims=True)), MIN_NORM)
    out_ref[...] = (_artanh(p_norm) / p_norm) * s


_dense = pl.pallas_call(
    _dense_body,
    grid=(N // ROW_BLK,),
    in_specs=[
        pl.BlockSpec((ROW_BLK, D), lambda i: (i, 0)),
        pl.BlockSpec((D, D), lambda i: (0, 0)),
        pl.BlockSpec((1, D), lambda i: (0, 0)),
    ],
    out_specs=pl.BlockSpec((ROW_BLK, D), lambda i: (i, 0)),
    out_shape=jax.ShapeDtypeStruct((N, D), jnp.float32),
)


def _sc_body(st_hbm, src_hbm, dst_hbm, zeros_hbm, out_hbm,
             src_v, dst_v, rows0, rows1, acc, gsem, ssem0, ssem1):
    cid = lax.axis_index("c")
    sid = lax.axis_index("s")
    row0 = sid * ROWS_PER_TILE
    pltpu.sync_copy(zeros_hbm, acc.at[pl.ds(row0, ROWS_PER_TILE)])
    plsc.subcore_barrier()

    rows = (rows0, rows1)
    ssems = (ssem0, ssem1)

    def gather(k, b):
        pltpu.async_copy(st_hbm.at[src_v.at[k]], rows[b], gsem).wait()

    def scatter_start(k, b):
        pltpu.async_copy(rows[b], acc.at[dst_v.at[k]], ssems[b], add=True)

    def scatter_wait(k, b):
        pltpu.make_async_copy(rows[b], acc.at[dst_v.at[k]], ssems[b]).wait()

    for blk in range(NBLK):
        pltpu.sync_copy(src_hbm.at[cid, sid, blk], src_v)
        pltpu.sync_copy(dst_hbm.at[cid, sid, blk], dst_v)

        gather(0, 0)
        scatter_start(0, 0)
        gather(1, 1)
        scatter_start(1, 1)

        def pair(i, carry):
            for b in range(2):
                k = 2 * i + b
                scatter_wait(k - 2, b)
                gather(k, b)
                scatter_start(k, b)
            return carry

        lax.fori_loop(1, KB // 2, pair, 0)
        scatter_wait(KB - 2, 0)
        scatter_wait(KB - 1, 1)

    plsc.subcore_barrier()
    pltpu.sync_copy(acc.at[pl.ds(row0, ROWS_PER_TILE)],
                    out_hbm.at[cid, pl.ds(row0, ROWS_PER_TILE)])


@functools.cache
def _sc_spmm():
    return pl.kernel(
        _sc_body,
        out_type=jax.ShapeDtypeStruct((NC, NPAD, D), jnp.float32),
        mesh=plsc.VectorSubcoreMesh(core_axis_name="c", subcore_axis_name="s",
                                    num_cores=NC, num_subcores=NS),
        scratch_types=[
            pltpu.VMEM((KB, CHUNK), jnp.int32),
            pltpu.VMEM((KB, CHUNK), jnp.int32),
            pltpu.VMEM((CHUNK, D), jnp.float32),
            pltpu.VMEM((CHUNK, D), jnp.float32),
            pltpu.VMEM_SHARED((NPAD, D), jnp.float32),
            pltpu.SemaphoreType.DMA,
            pltpu.SemaphoreType.DMA,
            pltpu.SemaphoreType.DMA,
        ],
    )


def _add_relu_body(p_ref, out_ref):
    out_ref[...] = jnp.maximum(p_ref[0] + p_ref[1], 0.0)


_add_relu = pl.pallas_call(
    _add_relu_body,
    grid=(N // ROW_BLK,),
    in_specs=[pl.BlockSpec((NC, ROW_BLK, D), lambda i: (0, i, 0))],
    out_specs=pl.BlockSpec((ROW_BLK, D), lambda i: (i, 0)),
    out_shape=jax.ShapeDtypeStruct((N, D), jnp.float32),
)


def kernel(adjacency_edge_index, input_feature, weight, bias):
    st = _dense(input_feature, weight, bias.reshape(1, D))
    ei = adjacency_edge_index.astype(jnp.int32)
    dst_i = ei[0].reshape(NC, NS, NBLK, KB, CHUNK)
    src_i = ei[1].reshape(NC, NS, NBLK, KB, CHUNK)
    zeros = jnp.zeros((ROWS_PER_TILE, D), jnp.float32)
    partials = _sc_spmm()(st, src_i, dst_i, zeros)
    return _add_relu(partials)
